# baseline (speedup 1.0000x reference)
import functools

import jax
import jax.numpy as jnp
from jax import lax
from jax.experimental import pallas as pl
from jax.experimental.pallas import tpu as pltpu
from jax.experimental.pallas import tpu_sc as plsc

B, S = 4096, 50
D = 300
DP = 304
DT = 38
BT = 32

NC, NS = 2, 16
NW = NC * NS
CB = 64

_mesh = plsc.VectorSubcoreMesh(core_axis_name="c", subcore_axis_name="s")


@functools.partial(
    pl.kernel,
    mesh=_mesh,
    out_type=jax.ShapeDtypeStruct((S * DT * BT * 8, 128), jnp.float32),
    scratch_types=[
        pltpu.VMEM((S, 128), jnp.int32),
        pltpu.VMEM((CB, DP), jnp.float32),
        pltpu.VMEM((CB, DP), jnp.float32),
        pltpu.VMEM((DT * 8, 128), jnp.float32),
        pltpu.SemaphoreType.DMA,
        pltpu.SemaphoreType.DMA,
        pltpu.SemaphoreType.DMA,
    ],
    compiler_params=pltpu.CompilerParams(needs_layout_passes=False,
                                         use_tc_tiling_on_sc=False),
)
def _emb_gather(idx_hbm, table_hbm, out_hbm, idx_v, bufa, bufb, tbuf,
                gsem0, gsem1, wsem):
    w = lax.axis_index("s") * NC + lax.axis_index("c")
    pltpu.sync_copy(idx_hbm.at[pl.ds(0, S), pl.ds(w * 128, 128)], idx_v)

    bufs = (bufa, bufb)
    gsems = (gsem0, gsem1)
    iota = jax.lax.iota(jnp.int32, 16)
    bio = [iota + 16 * g for g in range(CB // 16)]
    diag = [lax.bitwise_and(iota + k, 15) for k in range(16)]

    def fire_gathers(s, h, buf, gsem):
        pltpu.async_copy(table_hbm.at[idx_v.at[s, pl.ds(h * CB, CB)]],
                         buf, gsem)

    def drain_gathers(buf, gsem):
        pltpu.make_async_copy(table_hbm.at[pl.ds(0, CB)], buf, gsem).wait()

    def transpose(buf, h):
        lanes = [bio[g] + h * CB for g in range(CB // 16)]

        def d0_body(dblk, carry):
            d0 = dblk * 16
            for g in range(CB // 16):
                for k in range(16):
                    dcol = d0 + diag[k]
                    col = plsc.load_gather(buf, [bio[g], dcol])
                    plsc.store_scatter(tbuf, [dcol, lanes[g]], col)
            return carry
        lax.fori_loop(0, (DT * 8) // 16, d0_body, 0)

    def fire_write(s):
        base = s * (DT * BT * 8) + w * 8
        for dt in range(DT):
            pltpu.async_copy(tbuf.at[pl.ds(dt * 8, 8)],
                             out_hbm.at[pl.ds(base + dt * BT * 8, 8)], wsem)

    def drain_write():
        pltpu.make_async_copy(tbuf, out_hbm.at[pl.ds(0, DT * 8)], wsem).wait()

    fire_gathers(0, 0, bufa, gsem0)
    fire_gathers(0, 1, bufb, gsem1)

    def main_body(s, carry):
        drain_gathers(bufa, gsem0)

        @pl.when(s >= 1)
        def _():
            drain_write()
        transpose(bufa, 0)

        @pl.when(s <= S - 2)
        def _():
            fire_gathers(s + 1, 0, bufa, gsem0)
        drain_gathers(bufb, gsem1)
        transpose(bufb, 1)
        fire_write(s)

        @pl.when(s <= S - 2)
        def _():
            fire_gathers(s + 1, 1, bufb, gsem1)
        return carry

    lax.fori_loop(0, S, main_body, 0)
    drain_write()


def kernel(input, table):
    idx_t = input.astype(jnp.int32).T
    tpad = jnp.pad(table, ((0, 0), (0, DP - D)))
    z2 = _emb_gather(idx_t, tpad)
    z5 = z2.reshape(S, DT, BT, 8, 128)
    t = jnp.transpose(z5, (2, 4, 0, 1, 3))
    return t.reshape(B, S, DT * 8)[:, :, :D]


# speedup vs baseline: 1.4048x; 1.4048x over previous
import functools

import jax
import jax.numpy as jnp
from jax import lax
from jax.experimental import pallas as pl
from jax.experimental.pallas import tpu as pltpu
from jax.experimental.pallas import tpu_sc as plsc

B, S = 4096, 50
D = 300
DT = 38
BT = 32

NC, NS = 2, 16
NW = NC * NS
CB = 64

_mesh = plsc.VectorSubcoreMesh(core_axis_name="c", subcore_axis_name="s")


@functools.partial(
    pl.kernel,
    mesh=_mesh,
    out_type=jax.ShapeDtypeStruct((S * DT * BT * 8, 128), jnp.float32),
    scratch_types=[
        pltpu.VMEM((S, 128), jnp.int32),
        pltpu.VMEM((CB, D), jnp.float32),
        pltpu.VMEM((CB, D), jnp.float32),
        pltpu.VMEM((DT * 8, 128), jnp.float32),
        pltpu.SemaphoreType.DMA,
        pltpu.SemaphoreType.DMA,
        pltpu.SemaphoreType.DMA,
    ],
    compiler_params=pltpu.CompilerParams(needs_layout_passes=False),
)
def _emb_gather(idx_hbm, table_hbm, out_hbm, idx_v, bufa, bufb, tbuf,
                gsem0, gsem1, wsem):
    w = lax.axis_index("s") * NC + lax.axis_index("c")
    pltpu.sync_copy(idx_hbm.at[pl.ds(0, S), pl.ds(w * 128, 128)], idx_v)

    bufs = (bufa, bufb)
    gsems = (gsem0, gsem1)
    iota = jax.lax.iota(jnp.int32, 16)
    bio = [iota + 16 * g for g in range(CB // 16)]
    diag = [lax.bitwise_and(iota + k, 15) for k in range(16)]

    def fire_gathers(s, h, buf, gsem):
        def grp_body(g, carry):
            v = idx_v[s, pl.ds(h * CB + g * 16, 16)]
            for j in range(16):
                i = v[j]
                pltpu.async_copy(table_hbm.at[pl.ds(i, 1)],
                                 buf.at[pl.ds(g * 16 + j, 1)], gsem)
            return carry
        lax.fori_loop(0, CB // 16, grp_body, 0)

    def drain_gathers(buf, gsem):
        pltpu.make_async_copy(table_hbm.at[pl.ds(0, CB)], buf, gsem).wait()

    def transpose(buf, h):
        lanes = [bio[g] + h * CB for g in range(CB // 16)]

        def d0_body(dblk, carry):
            d0 = dblk * 16
            for g in range(CB // 16):
                for k in range(16):
                    dcol = d0 + diag[k]
                    col = plsc.load_gather(buf, [bio[g], dcol])
                    plsc.store_scatter(tbuf, [dcol, lanes[g]], col)
            return carry
        lax.fori_loop(0, (DT * 8) // 16, d0_body, 0)

    def fire_write(s):
        base = s * (DT * BT * 8) + w * 8
        for dt in range(DT):
            pltpu.async_copy(tbuf.at[pl.ds(dt * 8, 8)],
                             out_hbm.at[pl.ds(base + dt * BT * 8, 8)], wsem)

    def drain_write():
        pltpu.make_async_copy(tbuf, out_hbm.at[pl.ds(0, DT * 8)], wsem).wait()

    fire_gathers(0, 0, bufa, gsem0)
    fire_gathers(0, 1, bufb, gsem1)

    def main_body(s, carry):
        drain_gathers(bufa, gsem0)

        @pl.when(s >= 1)
        def _():
            drain_write()
        transpose(bufa, 0)

        @pl.when(s <= S - 2)
        def _():
            fire_gathers(s + 1, 0, bufa, gsem0)
        drain_gathers(bufb, gsem1)
        transpose(bufb, 1)
        fire_write(s)

        @pl.when(s <= S - 2)
        def _():
            fire_gathers(s + 1, 1, bufb, gsem1)
        return carry

    lax.fori_loop(0, S, main_body, 0)
    drain_write()


def kernel(input, table):
    idx_t = input.astype(jnp.int32).T
    z2 = _emb_gather(idx_t, table)
    z5 = z2.reshape(S, DT, BT, 8, 128)
    t = jnp.transpose(z5, (2, 4, 0, 1, 3))
    return t.reshape(B, S, DT * 8)[:, :, :D]
